# Initial kernel scaffold; baseline (speedup 1.0000x reference)
import jax
import jax.numpy as jnp
from jax.experimental import pallas as pl


def kernel(unknown, known, known_feats):
    raise NotImplementedError("write your pallas kernel here")


# speedup vs baseline: 42.8438x; 42.8438x over previous
import functools

import jax
import jax.numpy as jnp
from jax.experimental import pallas as pl
from jax.experimental.pallas import tpu as pltpu

_N_BLK = 512


def _fp_block_kernel(ux, uy, uz, kx, ky, kz, feats, out_ref):
    n_blk = ux.shape[-1]
    m = kx.shape[-1]
    d = (ux[0, 0, 0, :][:, None] - kx[0, 0, :][None, :]) ** 2
    d += (uy[0, 0, 0, :][:, None] - ky[0, 0, :][None, :]) ** 2
    d += (uz[0, 0, 0, :][:, None] - kz[0, 0, :][None, :]) ** 2

    iota = jax.lax.broadcasted_iota(jnp.int32, (n_blk, m), 1).astype(
        jnp.float32)
    dm = d
    w = jnp.zeros((n_blk, m), jnp.float32)
    norm = jnp.zeros((n_blk, 1), jnp.float32)
    for k in range(3):
        v = jnp.min(dm, axis=1, keepdims=True)
        idxf = jnp.min(jnp.where(dm == v, iota, float(m)), axis=1,
                       keepdims=True)
        oh = iota == idxf
        recip = 1.0 / (v + 1e-8)
        w = w + jnp.where(oh, recip, 0.0)
        norm = norm + recip
        if k < 2:
            dm = jnp.where(oh, jnp.inf, dm)
    w = w / norm

    out = jax.lax.dot_general(
        feats[0], w,
        dimension_numbers=(((1,), (1,)), ((), ())),
        preferred_element_type=jnp.float32,
    )
    out_ref[0] = out


@jax.jit
def kernel(unknown, known, known_feats):
    B, n, _ = unknown.shape
    _, m, _ = known.shape
    C = known_feats.shape[1]
    n_blk = _N_BLK

    ux, uy, uz = (unknown[:, :, i].reshape(B, n // n_blk, 1, n_blk)
                  for i in range(3))
    kx, ky, kz = (known[:, :, i].reshape(B, 1, m) for i in range(3))

    grid = (B, n // n_blk)
    u_spec = pl.BlockSpec((1, 1, 1, n_blk), lambda b, i: (b, i, 0, 0))
    k_spec = pl.BlockSpec((1, 1, m), lambda b, i: (b, 0, 0))
    f_spec = pl.BlockSpec((1, C, m), lambda b, i: (b, 0, 0))
    out_spec = pl.BlockSpec((1, C, n_blk), lambda b, i: (b, 0, i))

    return pl.pallas_call(
        _fp_block_kernel,
        grid=grid,
        in_specs=[u_spec, u_spec, u_spec, k_spec, k_spec, k_spec, f_spec],
        out_specs=out_spec,
        out_shape=jax.ShapeDtypeStruct((B, C, n), jnp.float32),
        compiler_params=pltpu.CompilerParams(
            dimension_semantics=("parallel", "arbitrary"),
        ),
    )(ux, uy, uz, kx, ky, kz, known_feats)


# speedup vs baseline: 60.6053x; 1.4146x over previous
import functools

import jax
import jax.numpy as jnp
from jax.experimental import pallas as pl
from jax.experimental.pallas import tpu as pltpu

_N_BLK = 512


def _fp_block_kernel(ux, uy, uz, kx, ky, kz, feats, out_ref):
    n_blk = ux.shape[-1]
    m = kx.shape[-1]
    d = (ux[0, 0, 0, :][:, None] - kx[0, 0, :][None, :]) ** 2
    d += (uy[0, 0, 0, :][:, None] - ky[0, 0, :][None, :]) ** 2
    d += (uz[0, 0, 0, :][:, None] - kz[0, 0, :][None, :]) ** 2

    v1 = jnp.min(d, axis=1, keepdims=True)
    d2 = jnp.where(d == v1, jnp.inf, d)
    v2 = jnp.min(d2, axis=1, keepdims=True)
    d3 = jnp.where(d2 == v2, jnp.inf, d2)
    v3 = jnp.min(d3, axis=1, keepdims=True)

    w = jnp.where(d <= v3, 1.0 / (d + 1e-8), 0.0)
    norm = jnp.sum(w, axis=1)

    out = jax.lax.dot_general(
        feats[0], w,
        dimension_numbers=(((1,), (1,)), ((), ())),
        preferred_element_type=jnp.float32,
    )
    out_ref[0] = out * (1.0 / norm)[None, :]


@jax.jit
def kernel(unknown, known, known_feats):
    B, n, _ = unknown.shape
    _, m, _ = known.shape
    C = known_feats.shape[1]
    n_blk = _N_BLK

    ux, uy, uz = (unknown[:, :, i].reshape(B, n // n_blk, 1, n_blk)
                  for i in range(3))
    kx, ky, kz = (known[:, :, i].reshape(B, 1, m) for i in range(3))

    grid = (B, n // n_blk)
    u_spec = pl.BlockSpec((1, 1, 1, n_blk), lambda b, i: (b, i, 0, 0))
    k_spec = pl.BlockSpec((1, 1, m), lambda b, i: (b, 0, 0))
    f_spec = pl.BlockSpec((1, C, m), lambda b, i: (b, 0, 0))
    out_spec = pl.BlockSpec((1, C, n_blk), lambda b, i: (b, 0, i))

    return pl.pallas_call(
        _fp_block_kernel,
        grid=grid,
        in_specs=[u_spec, u_spec, u_spec, k_spec, k_spec, k_spec, f_spec],
        out_specs=out_spec,
        out_shape=jax.ShapeDtypeStruct((B, C, n), jnp.float32),
        compiler_params=pltpu.CompilerParams(
            dimension_semantics=("parallel", "arbitrary"),
        ),
    )(ux, uy, uz, kx, ky, kz, known_feats)
